# Initial kernel scaffold; baseline (speedup 1.0000x reference)
import jax
import jax.numpy as jnp
from jax.experimental import pallas as pl


def kernel(exercises, categories, exercise_table, category_table, position_table):
    raise NotImplementedError("write your pallas kernel here")


# speedup vs baseline: 3.5588x; 3.5588x over previous
import functools

import jax
import jax.numpy as jnp
from jax import lax
from jax.experimental import pallas as pl
from jax.experimental.pallas import tpu as pltpu
from jax.experimental.pallas import tpu_sc as plsc

N_EX = 100000
N_CAT = 1000
D = 64
S = 200
B = 4096

_NC = 2
_NS = 16
_NW = _NC * _NS
_ROWS_PER_W = B // _NW
_LANES = 16
_VECS_PER_ROW = D // _LANES

_GATHER_SPLITS = ((0, 128), (128, 72))


def _emb_body(ex_hbm, cat_hbm, etab_hbm, ctab_hbm, ptab_hbm, out_hbm,
              idx_e, idx_c, e_buf, c_buf, o_buf, p_buf, sem):
    wid = lax.axis_index("s") * _NC + lax.axis_index("c")
    base = wid * _ROWS_PER_W

    pltpu.sync_copy(ptab_hbm, p_buf)

    def row_body(r, carry):
        row = base + r
        pltpu.sync_copy(ex_hbm.at[row], idx_e)
        pltpu.sync_copy(cat_hbm.at[row], idx_c)
        cps = []
        for idx, tab, buf in ((idx_e, etab_hbm, e_buf), (idx_c, ctab_hbm, c_buf)):
            for lo, ln in _GATHER_SPLITS:
                cps.append(pltpu.async_copy(
                    tab.at[idx.at[pl.ds(lo, ln)]],
                    buf.at[pl.ds(lo, ln)],
                    sem,
                ))
        for cp in cps:
            cp.wait()

        def s_body(s, c2):
            for j in range(_VECS_PER_ROW):
                sl = pl.ds(j * _LANES, _LANES)
                o_buf[s, sl] = e_buf[s, sl] + c_buf[s, sl] + p_buf[s, sl]
            return c2

        lax.fori_loop(0, S, s_body, 0, unroll=2)
        pltpu.sync_copy(o_buf, out_hbm.at[row])
        return carry

    lax.fori_loop(0, _ROWS_PER_W, row_body, 0)


_emb_kernel = functools.partial(
    pl.kernel,
    out_type=jax.ShapeDtypeStruct((B, S, D), jnp.float32),
    scratch_types=[
        pltpu.VMEM((S,), jnp.int32),
        pltpu.VMEM((S,), jnp.int32),
        pltpu.VMEM((S, D), jnp.float32),
        pltpu.VMEM((S, D), jnp.float32),
        pltpu.VMEM((S, D), jnp.float32),
        pltpu.VMEM((S, D), jnp.float32),
        pltpu.SemaphoreType.DMA,
    ],
    mesh=plsc.VectorSubcoreMesh(core_axis_name="c", subcore_axis_name="s"),
    compiler_params=pltpu.CompilerParams(use_tc_tiling_on_sc=False),
)(_emb_body)


def kernel(exercises, categories, exercise_table, category_table, position_table):
    return _emb_kernel(exercises, categories.astype(jnp.int32),
                       exercise_table, category_table, position_table)


# speedup vs baseline: 4.7496x; 1.3346x over previous
import functools

import jax
import jax.numpy as jnp
from jax import lax
from jax.experimental import pallas as pl
from jax.experimental.pallas import tpu as pltpu
from jax.experimental.pallas import tpu_sc as plsc

N_EX = 100000
N_CAT = 1000
D = 64
S = 200
B = 4096

_NC = 2
_NS = 16
_NW = _NC * _NS
_ROWS_PER_W = B // _NW
_HALF = 64
_LANES = 16
_VECS_PER_ROW = D // _LANES

_GATHER_SPLITS = ((0, 128), (128, 72))


def _emb_body(ex_hbm, cat_hbm, etab_hbm, ctab_hbm, ptab_hbm, out_hbm,
              idx_e, idx_c, e_buf, c_buf, o_buf, p_buf,
              sem_g0, sem_g1, sem_o0, sem_o1):
    wid = lax.axis_index("s") * _NC + lax.axis_index("c")
    base = wid * _ROWS_PER_W
    sem_g = (sem_g0, sem_g1)
    sem_o = (sem_o0, sem_o1)

    pltpu.sync_copy(ptab_hbm, p_buf)

    def gcopies(rr, k):
        cps = []
        for idx, tab, buf in ((idx_e, etab_hbm, e_buf), (idx_c, ctab_hbm, c_buf)):
            for lo, ln in _GATHER_SPLITS:
                cps.append(pltpu.make_async_copy(
                    tab.at[idx.at[rr, pl.ds(lo, ln)]],
                    buf.at[k, pl.ds(lo, ln)],
                    sem_g[k],
                ))
        return cps

    def half_body(h, carry):
        row0 = base + h * _HALF
        pltpu.sync_copy(ex_hbm.at[pl.ds(row0, _HALF)], idx_e)
        pltpu.sync_copy(cat_hbm.at[pl.ds(row0, _HALF)], idx_c)

        for cp in gcopies(0, 0):
            cp.start()

        def pair_body(g, c1):
            for k in range(2):
                rr = 2 * g + k
                for cp in gcopies(rr, k):
                    cp.wait()

                @pl.when(rr + 1 < _HALF)
                def _():
                    for cp in gcopies(rr + 1, k ^ 1):
                        cp.start()

                @pl.when(rr >= 2)
                def _():
                    pltpu.make_async_copy(
                        o_buf.at[k], out_hbm.at[row0 + rr - 2], sem_o[k]).wait()

                def s_body(s, c2):
                    for j in range(_VECS_PER_ROW):
                        sl = pl.ds(j * _LANES, _LANES)
                        o_buf[k, s, sl] = e_buf[k, s, sl] + c_buf[k, s, sl] + p_buf[s, sl]
                    return c2

                lax.fori_loop(0, S, s_body, 0, unroll=2)
                pltpu.make_async_copy(
                    o_buf.at[k], out_hbm.at[row0 + rr], sem_o[k]).start()
            return c1

        lax.fori_loop(0, _HALF // 2, pair_body, 0)
        pltpu.make_async_copy(
            o_buf.at[0], out_hbm.at[row0 + _HALF - 2], sem_o[0]).wait()
        pltpu.make_async_copy(
            o_buf.at[1], out_hbm.at[row0 + _HALF - 1], sem_o[1]).wait()
        return carry

    lax.fori_loop(0, _ROWS_PER_W // _HALF, half_body, 0)


_emb_kernel = functools.partial(
    pl.kernel,
    out_type=jax.ShapeDtypeStruct((B, S, D), jnp.float32),
    scratch_types=[
        pltpu.VMEM((_HALF, S), jnp.int32),
        pltpu.VMEM((_HALF, S), jnp.int32),
        pltpu.VMEM((2, S, D), jnp.float32),
        pltpu.VMEM((2, S, D), jnp.float32),
        pltpu.VMEM((2, S, D), jnp.float32),
        pltpu.VMEM((S, D), jnp.float32),
        pltpu.SemaphoreType.DMA,
        pltpu.SemaphoreType.DMA,
        pltpu.SemaphoreType.DMA,
        pltpu.SemaphoreType.DMA,
    ],
    mesh=plsc.VectorSubcoreMesh(core_axis_name="c", subcore_axis_name="s"),
    compiler_params=pltpu.CompilerParams(use_tc_tiling_on_sc=False),
)(_emb_body)


def kernel(exercises, categories, exercise_table, category_table, position_table):
    return _emb_kernel(exercises, categories.astype(jnp.int32),
                       exercise_table, category_table, position_table)


# speedup vs baseline: 5.7401x; 1.2085x over previous
import functools

import jax
import jax.numpy as jnp
from jax import lax
from jax.experimental import pallas as pl
from jax.experimental.pallas import tpu as pltpu
from jax.experimental.pallas import tpu_sc as plsc

N_EX = 100000
N_CAT = 1000
D = 64
S = 200
B = 4096
S2 = 2 * S

_NC = 2
_NS = 16
_NW = _NC * _NS
_ROWS_PER_W = B // _NW
_NSLOT = 4
_LANES = 16
_VECS_PER_ROW = D // _LANES

_GATHER_SPLITS = ((0, 128), (128, 128), (256, 128), (384, 16))


def _emb_body(idx_hbm, tab_hbm, ptab_hbm, out_hbm,
              idx_v, d_buf, p_buf, *sems):
    sem_i = sems[0:_NSLOT]
    sem_g = sems[_NSLOT:2 * _NSLOT]
    sem_o = sems[2 * _NSLOT:3 * _NSLOT]
    wid = lax.axis_index("s") * _NC + lax.axis_index("c")
    base = wid * _ROWS_PER_W

    pltpu.sync_copy(ptab_hbm, p_buf)

    def idx_cp(row, k):
        return pltpu.make_async_copy(idx_hbm.at[row], idx_v.at[k], sem_i[k])

    def gather_cps(k):
        return [pltpu.make_async_copy(
                    tab_hbm.at[idx_v.at[k, pl.ds(lo, ln)]],
                    d_buf.at[k, pl.ds(lo, ln)],
                    sem_g[k])
                for lo, ln in _GATHER_SPLITS]

    def out_cp(row, k):
        return pltpu.make_async_copy(
            d_buf.at[k, pl.ds(0, S)], out_hbm.at[row], sem_o[k])

    idx_cp(base, 0).start()
    idx_cp(base + 1, 1).start()
    idx_cp(base, 0).wait()
    for cp in gather_cps(0):
        cp.start()

    def group_body(g, carry):
        for k in range(_NSLOT):
            r = _NSLOT * g + k
            k2 = (k + 1) % _NSLOT
            k3 = (k + 2) % _NSLOT

            for cp in gather_cps(k):
                cp.wait()

            @pl.when(r + 1 < _ROWS_PER_W)
            def _():
                idx_cp(base + r + 1, k2).wait()

                @pl.when(r >= _NSLOT - 1)
                def _():
                    out_cp(base + r - (_NSLOT - 1), k2).wait()

                for cp in gather_cps(k2):
                    cp.start()

            @pl.when(r + 2 < _ROWS_PER_W)
            def _():
                idx_cp(base + r + 2, k3).start()

            def s_body(s, c2):
                for j in range(_VECS_PER_ROW):
                    sl = pl.ds(j * _LANES, _LANES)
                    d_buf[k, s, sl] = d_buf[k, s, sl] + d_buf[k, S + s, sl] + p_buf[s, sl]
                return c2

            lax.fori_loop(0, S, s_body, 0, unroll=2)
            out_cp(base + r, k).start()
        return carry

    lax.fori_loop(0, _ROWS_PER_W // _NSLOT, group_body, 0)

    for r in range(_ROWS_PER_W - _NSLOT + 1, _ROWS_PER_W):
        out_cp(base + r, r % _NSLOT).wait()


_emb_kernel = functools.partial(
    pl.kernel,
    out_type=jax.ShapeDtypeStruct((B, S, D), jnp.float32),
    scratch_types=(
        [pltpu.VMEM((_NSLOT, S2), jnp.int32),
         pltpu.VMEM((_NSLOT, S2, D), jnp.float32),
         pltpu.VMEM((S, D), jnp.float32)]
        + [pltpu.SemaphoreType.DMA] * (3 * _NSLOT)
    ),
    mesh=plsc.VectorSubcoreMesh(core_axis_name="c", subcore_axis_name="s"),
    compiler_params=pltpu.CompilerParams(use_tc_tiling_on_sc=False),
)(_emb_body)


def kernel(exercises, categories, exercise_table, category_table, position_table):
    idx = jnp.concatenate(
        [exercises, categories.astype(jnp.int32) + N_EX], axis=1)
    tab = jnp.concatenate([exercise_table, category_table], axis=0)
    return _emb_kernel(idx, tab, position_table)


# speedup vs baseline: 5.7837x; 1.0076x over previous
import functools

import jax
import jax.numpy as jnp
from jax import lax
from jax.experimental import pallas as pl
from jax.experimental.pallas import tpu as pltpu
from jax.experimental.pallas import tpu_sc as plsc

N_EX = 100000
N_CAT = 1000
D = 64
S = 200
B = 4096
S2 = 2 * S

_NC = 2
_NS = 16
_NW = _NC * _NS
_ROWS_PER_W = B // _NW
_NSLOT = 4
_NIDX = 4
_LANES = 16
_VECS_PER_ROW = D // _LANES

_GATHER_SPLITS = ((0, 128), (128, 128), (256, 128), (384, 16))


def _emb_body(idx_hbm, tab_hbm, ptab_hbm, out_hbm,
              idx_v, d_buf, p_buf, *sems):
    sem_i = sems[0:_NIDX]
    sem_g = sems[_NIDX:_NIDX + _NSLOT]
    sem_o = sems[_NIDX + _NSLOT:_NIDX + 2 * _NSLOT]
    wid = lax.axis_index("s") * _NC + lax.axis_index("c")
    base = wid * _ROWS_PER_W

    pltpu.sync_copy(ptab_hbm, p_buf)

    def idx_cp(row, ki):
        return pltpu.make_async_copy(idx_hbm.at[row], idx_v.at[ki], sem_i[ki])

    def gather_cps(ki, kd):
        return [pltpu.make_async_copy(
                    tab_hbm.at[idx_v.at[ki, pl.ds(lo, ln)]],
                    d_buf.at[kd, pl.ds(lo, ln)],
                    sem_g[kd])
                for lo, ln in _GATHER_SPLITS]

    def out_cp(row, kd):
        return pltpu.make_async_copy(
            d_buf.at[kd, pl.ds(0, S)], out_hbm.at[row], sem_o[kd])

    for r in range(3):
        idx_cp(base + r, r).start()
    for r in range(2):
        idx_cp(base + r, r).wait()
        for cp in gather_cps(r, r):
            cp.start()

    def group_body(g, carry):
        for k in range(_NSLOT):
            r = _NSLOT * g + k
            kd2 = (k + 2) % _NSLOT

            for cp in gather_cps(0, k):
                cp.wait()

            @pl.when(r + 2 < _ROWS_PER_W)
            def _():
                idx_cp(base + r + 2, (k + 2) % _NIDX).wait()

                @pl.when(r >= 2)
                def _():
                    out_cp(base + r - 2, kd2).wait()

                for cp in gather_cps((k + 2) % _NIDX, kd2):
                    cp.start()

            @pl.when(r + 3 < _ROWS_PER_W)
            def _():
                idx_cp(base + r + 3, (k + 3) % _NIDX).start()

            def s_body(s, c2):
                for j in range(_VECS_PER_ROW):
                    sl = pl.ds(j * _LANES, _LANES)
                    d_buf[k, s, sl] = d_buf[k, s, sl] + d_buf[k, S + s, sl] + p_buf[s, sl]
                return c2

            lax.fori_loop(0, S, s_body, 0, unroll=2)
            out_cp(base + r, k).start()
        return carry

    lax.fori_loop(0, _ROWS_PER_W // _NSLOT, group_body, 0)

    for r in range(_ROWS_PER_W - 4, _ROWS_PER_W):
        out_cp(base + r, r % _NSLOT).wait()


_emb_kernel = functools.partial(
    pl.kernel,
    out_type=jax.ShapeDtypeStruct((B, S, D), jnp.float32),
    scratch_types=(
        [pltpu.VMEM((_NIDX, S2), jnp.int32),
         pltpu.VMEM((_NSLOT, S2, D), jnp.float32),
         pltpu.VMEM((S, D), jnp.float32)]
        + [pltpu.SemaphoreType.DMA] * (_NIDX + 2 * _NSLOT)
    ),
    mesh=plsc.VectorSubcoreMesh(core_axis_name="c", subcore_axis_name="s"),
    compiler_params=pltpu.CompilerParams(use_tc_tiling_on_sc=False),
)(_emb_body)


def kernel(exercises, categories, exercise_table, category_table, position_table):
    idx = jnp.concatenate(
        [exercises, categories.astype(jnp.int32) + N_EX], axis=1)
    tab = jnp.concatenate([exercise_table, category_table], axis=0)
    return _emb_kernel(idx, tab, position_table)
